# Initial kernel scaffold; baseline (speedup 1.0000x reference)
import jax
import jax.numpy as jnp
from jax.experimental import pallas as pl


def kernel(inputs, tables):
    raise NotImplementedError("write your pallas kernel here")


# speedup vs baseline: 2.0816x; 2.0816x over previous
import functools

import jax
import jax.numpy as jnp
from jax import lax
from jax.experimental import pallas as pl
from jax.experimental.pallas import tpu as pltpu
from jax.experimental.pallas import tpu_sc as plsc

B = 16384
F = 26
V = 64
D = 32
L = 16

CHUNK = 128
CHUNK_W = CHUNK * F
DBLK = 8
TABLE_W = F * V * D


def _sc_embed(jbase, tables_flat, nw):
    chunks_per_w = B // CHUNK // nw
    mesh = plsc.VectorSubcoreMesh(core_axis_name="c", subcore_axis_name="s")

    @functools.partial(
        pl.kernel,
        mesh=mesh,
        out_type=jax.ShapeDtypeStruct((D, B * F), jnp.float32),
        scratch_types=[
            pltpu.VMEM((TABLE_W,), jnp.float32),
            pltpu.VMEM((CHUNK_W,), jnp.int32),
            pltpu.VMEM((DBLK * CHUNK_W,), jnp.float32),
            pltpu.SemaphoreType.DMA,
            pltpu.SemaphoreType.DMA,
        ],
        compiler_params=pltpu.CompilerParams(needs_layout_passes=False),
    )
    def k(jbase_hbm, tab_hbm, out_hbm, tab_v, idx_v, out_v, sem_in, sem_out):
        wid = lax.axis_index("s") * 2 + lax.axis_index("c")
        pltpu.async_copy(tab_hbm, tab_v, sem_in).wait()

        for c in range(chunks_per_w):
            b0 = (wid * chunks_per_w + c) * CHUNK
            off = b0 * F
            pltpu.async_copy(
                jbase_hbm.at[pl.ds(off, CHUNK_W)], idx_v, sem_in
            ).wait()
            for p in range(D // DBLK):
                dlo = p * DBLK

                def body(g, _):
                    jb = idx_v[pl.ds(g * L, L)] + dlo
                    for dd in range(DBLK):
                        v = plsc.load_gather(tab_v, [jb])
                        out_v[pl.ds(dd * CHUNK_W + g * L, L)] = v
                        if dd + 1 < DBLK:
                            jb = jb + 1
                    return 0

                lax.fori_loop(0, CHUNK_W // L, body, 0)
                copies = [
                    pltpu.async_copy(
                        out_v.at[pl.ds(dd * CHUNK_W, CHUNK_W)],
                        out_hbm.at[dlo + dd, pl.ds(off, CHUNK_W)],
                        sem_out,
                    )
                    for dd in range(DBLK)
                ]
                for cp in copies:
                    cp.wait()

    return k(jbase, tables_flat)


def kernel(inputs, tables):
    jbase = (inputs.astype(jnp.int32) * D
             + (jnp.arange(F, dtype=jnp.int32) * (V * D))[None, :])
    jbase = jbase.reshape(B * F)
    tables_flat = tables.reshape(TABLE_W)
    out2 = _sc_embed(jbase, tables_flat, 32)
    return out2.reshape(D, B, F)


# speedup vs baseline: 2.4486x; 1.1763x over previous
import functools

import jax
import jax.numpy as jnp
from jax import lax
from jax.experimental import pallas as pl
from jax.experimental.pallas import tpu as pltpu
from jax.experimental.pallas import tpu_sc as plsc

B = 16384
F = 26
V = 64
D = 32
L = 16

CHUNK = 128
CHUNK_W = CHUNK * F
DBLK = 8
TABLE_W = F * V * D


def _sc_embed(jbase, tables_flat, nw):
    chunks_per_w = B // CHUNK // nw
    mesh = plsc.VectorSubcoreMesh(core_axis_name="c", subcore_axis_name="s")

    @functools.partial(
        pl.kernel,
        mesh=mesh,
        out_type=jax.ShapeDtypeStruct((D, B * F), jnp.float32),
        scratch_types=[
            pltpu.VMEM((TABLE_W,), jnp.float32),
            pltpu.VMEM((CHUNK_W,), jnp.int32),
            pltpu.VMEM((DBLK * CHUNK_W,), jnp.float32),
            pltpu.SemaphoreType.DMA,
            pltpu.SemaphoreType.DMA,
        ],
        compiler_params=pltpu.CompilerParams(needs_layout_passes=False),
    )
    def k(jbase_hbm, tab_hbm, out_hbm, tab_v, idx_v, out_v, sem_in, sem_out):
        wid = lax.axis_index("s") * 2 + lax.axis_index("c")
        pltpu.async_copy(tab_hbm, tab_v, sem_in).wait()

        for c in range(chunks_per_w):
            b0 = (wid * chunks_per_w + c) * CHUNK
            off = b0 * F
            pltpu.async_copy(
                jbase_hbm.at[pl.ds(off, CHUNK_W)], idx_v, sem_in
            ).wait()
            for p in range(D // DBLK):
                dlo = p * DBLK

                @plsc.parallel_loop(0, CHUNK_W, L, unroll=4)
                def body(g):
                    jb = idx_v[pl.ds(g, L)] + dlo
                    for dd in range(DBLK):
                        v = plsc.load_gather(tab_v, [jb])
                        out_v[pl.ds(dd * CHUNK_W + g, L)] = v
                        if dd + 1 < DBLK:
                            jb = jb + 1
                copies = [
                    pltpu.async_copy(
                        out_v.at[pl.ds(dd * CHUNK_W, CHUNK_W)],
                        out_hbm.at[dlo + dd, pl.ds(off, CHUNK_W)],
                        sem_out,
                    )
                    for dd in range(DBLK)
                ]
                for cp in copies:
                    cp.wait()

    return k(jbase, tables_flat)


def kernel(inputs, tables):
    jbase = (inputs.astype(jnp.int32) * D
             + (jnp.arange(F, dtype=jnp.int32) * (V * D))[None, :])
    jbase = jbase.reshape(B * F)
    tables_flat = tables.reshape(TABLE_W)
    out2 = _sc_embed(jbase, tables_flat, 32)
    return out2.reshape(D, B, F)


# speedup vs baseline: 3.5860x; 1.4645x over previous
import functools

import jax
import jax.numpy as jnp
from jax import lax
from jax.experimental import pallas as pl
from jax.experimental.pallas import tpu as pltpu
from jax.experimental.pallas import tpu_sc as plsc

B = 16384
F = 26
V = 64
D = 32
L = 16

CHUNK = 64
CHUNK_W = CHUNK * F
DBLK = 8
TABLE_W = F * V * D


def _sc_embed(jbase, tables_flat, nw):
    chunks_per_w = B // CHUNK // nw
    mesh = plsc.VectorSubcoreMesh(core_axis_name="c", subcore_axis_name="s")

    @functools.partial(
        pl.kernel,
        mesh=mesh,
        out_type=jax.ShapeDtypeStruct((D, B, F), jnp.float32),
        scratch_types=[
            pltpu.VMEM((TABLE_W,), jnp.float32),
            pltpu.VMEM((CHUNK_W,), jnp.int32),
            pltpu.VMEM((DBLK, CHUNK, F), jnp.float32),
            pltpu.SemaphoreType.DMA,
            pltpu.SemaphoreType.DMA,
        ],
        compiler_params=pltpu.CompilerParams(needs_layout_passes=False),
    )
    def k(jbase_hbm, tab_hbm, out_hbm, tab_v, idx_v, out_v, sem_in, sem_out):
        wid = lax.axis_index("s") * 2 + lax.axis_index("c")
        pltpu.async_copy(tab_hbm, tab_v, sem_in).wait()

        def chunk_body(c, _):
            b0 = (wid * chunks_per_w + c) * CHUNK
            pltpu.async_copy(
                jbase_hbm.at[pl.ds(b0 * F, CHUNK_W)], idx_v, sem_in
            ).wait()
            for p in range(D // DBLK):
                dlo = p * DBLK

                @plsc.parallel_loop(0, CHUNK, 1, unroll=2)
                def body(b):
                    ja = idx_v[pl.ds(b * F, L)] + dlo
                    jb = idx_v[pl.ds(b * F + (F - L), L)] + dlo
                    for dd in range(DBLK):
                        va = plsc.load_gather(tab_v, [ja])
                        vb = plsc.load_gather(tab_v, [jb])
                        out_v[dd, b, pl.ds(0, L)] = va
                        out_v[dd, b, pl.ds(F - L, L)] = vb
                        if dd + 1 < DBLK:
                            ja = ja + 1
                            jb = jb + 1

                copies = [
                    pltpu.async_copy(
                        out_v.at[dd],
                        out_hbm.at[dlo + dd, pl.ds(b0, CHUNK), :],
                        sem_out,
                    )
                    for dd in range(DBLK)
                ]
                for cp in copies:
                    cp.wait()
            return 0

        lax.fori_loop(0, chunks_per_w, chunk_body, 0)

    return k(jbase, tables_flat)


def kernel(inputs, tables):
    jbase = (inputs.astype(jnp.int32) * D
             + (jnp.arange(F, dtype=jnp.int32) * (V * D))[None, :])
    jbase = jbase.reshape(B * F)
    tables_flat = tables.reshape(TABLE_W)
    return _sc_embed(jbase, tables_flat, 32)


# speedup vs baseline: 3.5896x; 1.0010x over previous
import functools

import jax
import jax.numpy as jnp
from jax import lax
from jax.experimental import pallas as pl
from jax.experimental.pallas import tpu as pltpu
from jax.experimental.pallas import tpu_sc as plsc

B = 16384
F = 26
V = 64
D = 32
L = 16

CHUNK = 64
CHUNK_W = CHUNK * F
DBLK = 8
TABLE_W = F * V * D


def _sc_embed(jbase, tables_flat, nw):
    chunks_per_w = B // CHUNK // nw
    mesh = plsc.VectorSubcoreMesh(core_axis_name="c", subcore_axis_name="s")

    @functools.partial(
        pl.kernel,
        mesh=mesh,
        out_type=jax.ShapeDtypeStruct((D, B, F), jnp.float32),
        scratch_types=[
            pltpu.VMEM((TABLE_W,), jnp.float32),
            pltpu.VMEM((CHUNK_W,), jnp.int32),
            pltpu.VMEM((DBLK, CHUNK, F), jnp.float32),
            pltpu.SemaphoreType.DMA,
            pltpu.SemaphoreType.DMA,
        ],
        compiler_params=pltpu.CompilerParams(
            needs_layout_passes=False, use_tc_tiling_on_sc=True
        ),
    )
    def k(jbase_hbm, tab_hbm, out_hbm, tab_v, idx_v, out_v, sem_in, sem_out):
        wid = lax.axis_index("s") * 2 + lax.axis_index("c")
        pltpu.async_copy(tab_hbm, tab_v, sem_in).wait()

        def chunk_body(c, _):
            b0 = (wid * chunks_per_w + c) * CHUNK
            pltpu.async_copy(
                jbase_hbm.at[pl.ds(b0 * F, CHUNK_W)], idx_v, sem_in
            ).wait()
            for p in range(D // DBLK):
                dlo = p * DBLK

                @plsc.parallel_loop(0, CHUNK, 1, unroll=2)
                def body(b):
                    ja = idx_v[pl.ds(b * F, L)] + dlo
                    jb = idx_v[pl.ds(b * F + (F - L), L)] + dlo
                    for dd in range(DBLK):
                        va = plsc.load_gather(tab_v, [ja])
                        vb = plsc.load_gather(tab_v, [jb])
                        out_v[dd, b, pl.ds(0, L)] = va
                        out_v[dd, b, pl.ds(F - L, L)] = vb
                        if dd + 1 < DBLK:
                            ja = ja + 1
                            jb = jb + 1

                copies = [
                    pltpu.async_copy(
                        out_v.at[dd],
                        out_hbm.at[dlo + dd, pl.ds(b0, CHUNK), :],
                        sem_out,
                    )
                    for dd in range(DBLK)
                ]
                for cp in copies:
                    cp.wait()
            return 0

        lax.fori_loop(0, chunks_per_w, chunk_body, 0)

    return k(jbase, tables_flat)


def kernel(inputs, tables):
    jbase = (inputs.astype(jnp.int32) * D
             + (jnp.arange(F, dtype=jnp.int32) * (V * D))[None, :])
    jbase = jbase.reshape(B * F)
    tables_flat = tables.reshape(TABLE_W)
    return _sc_embed(jbase, tables_flat, 32)


# speedup vs baseline: 4.4153x; 1.2300x over previous
import functools

import jax
import jax.numpy as jnp
from jax import lax
from jax.experimental import pallas as pl
from jax.experimental.pallas import tpu as pltpu
from jax.experimental.pallas import tpu_sc as plsc

B = 16384
F = 26
V = 64
D = 32
L = 16

CHUNK = 64
CHUNK_W = CHUNK * F
DBLK = 8
TABLE_W = F * V * D


def _sc_embed(jbase, tables_flat, nw):
    chunks_per_w = B // CHUNK // nw
    mesh = plsc.VectorSubcoreMesh(core_axis_name="c", subcore_axis_name="s")

    @functools.partial(
        pl.kernel,
        mesh=mesh,
        out_type=jax.ShapeDtypeStruct((D * B, F), jnp.float32),
        scratch_types=[
            pltpu.VMEM((TABLE_W,), jnp.float32),
            pltpu.VMEM((CHUNK_W,), jnp.int32),
            pltpu.VMEM((DBLK, CHUNK, F), jnp.float32),
            pltpu.SemaphoreType.DMA,
            pltpu.SemaphoreType.DMA,
        ],
        compiler_params=pltpu.CompilerParams(needs_layout_passes=False),
    )
    def k(jbase_hbm, tab_hbm, out_hbm, tab_v, idx_v, out_v, sem_in, sem_out):
        wid = lax.axis_index("s") * 2 + lax.axis_index("c")
        pltpu.async_copy(tab_hbm, tab_v, sem_in).wait()

        def chunk_body(c, _):
            b0 = (wid * chunks_per_w + c) * CHUNK
            pltpu.async_copy(
                jbase_hbm.at[pl.ds(b0 * F, CHUNK_W)], idx_v, sem_in
            ).wait()
            for p in range(D // DBLK):
                dlo = p * DBLK

                @plsc.parallel_loop(0, CHUNK, 1, unroll=2)
                def body(b):
                    ja = idx_v[pl.ds(b * F, L)] + dlo
                    jb = idx_v[pl.ds(b * F + (F - L), L)] + dlo
                    for dd in range(DBLK):
                        va = plsc.load_gather(tab_v, [ja])
                        vb = plsc.load_gather(tab_v, [jb])
                        out_v[dd, b, pl.ds(0, L)] = va
                        out_v[dd, b, pl.ds(F - L, L)] = vb
                        if dd + 1 < DBLK:
                            ja = ja + 1
                            jb = jb + 1

                copies = [
                    pltpu.async_copy(
                        out_v.at[dd],
                        out_hbm.at[pl.ds((dlo + dd) * B + b0, CHUNK), :],
                        sem_out,
                    )
                    for dd in range(DBLK)
                ]
                for cp in copies:
                    cp.wait()
            return 0

        lax.fori_loop(0, chunks_per_w, chunk_body, 0)

    return k(jbase, tables_flat)


def kernel(inputs, tables):
    jbase = (inputs.astype(jnp.int32) * D
             + (jnp.arange(F, dtype=jnp.int32) * (V * D))[None, :])
    jbase = jbase.reshape(B * F)
    tables_flat = tables.reshape(TABLE_W)
    out2 = _sc_embed(jbase, tables_flat, 32)
    return out2.reshape(D, B, F)


# speedup vs baseline: 8.8175x; 1.9970x over previous
import functools

import jax
import jax.numpy as jnp
from jax import lax
from jax.experimental import pallas as pl
from jax.experimental.pallas import tpu as pltpu
from jax.experimental.pallas import tpu_sc as plsc

B = 16384
F = 26
V = 64
D = 32
L = 16

CHUNK = 64
CHUNK_W = CHUNK * F
DBLK = 8
TABLE_W = F * V * D


def _sc_embed(jbase, tables_flat, nw):
    chunks_per_w = B // CHUNK // nw
    mesh = plsc.VectorSubcoreMesh(core_axis_name="c", subcore_axis_name="s")

    @functools.partial(
        pl.kernel,
        mesh=mesh,
        out_type=jax.ShapeDtypeStruct((D * B, F), jnp.float32),
        scratch_types=[
            pltpu.VMEM((TABLE_W,), jnp.float32),
            pltpu.VMEM((CHUNK_W,), jnp.int32),
            pltpu.VMEM((DBLK, CHUNK, F), jnp.float32),
            pltpu.SemaphoreType.DMA,
            pltpu.SemaphoreType.DMA,
        ],
        compiler_params=pltpu.CompilerParams(needs_layout_passes=False),
    )
    def k(jbase_hbm, tab_hbm, out_hbm, tab_v, idx_v, out_v, sem_in, sem_out):
        wid = lax.axis_index("s") * 2 + lax.axis_index("c")
        pltpu.async_copy(tab_hbm, tab_v, sem_in).wait()

        def chunk_body(c, _):
            b0 = (wid * chunks_per_w + c) * CHUNK
            pltpu.async_copy(
                jbase_hbm.at[pl.ds(b0 * F, CHUNK_W)], idx_v, sem_in
            ).wait()
            for p in range(D // DBLK):
                dlo = p * DBLK

                @plsc.parallel_loop(0, CHUNK, 1, unroll=2)
                def body(b):
                    ja = idx_v[pl.ds(b * F, L)] + dlo * (F * V)
                    jb = idx_v[pl.ds(b * F + (F - L), L)] + dlo * (F * V)
                    for dd in range(DBLK):
                        va = plsc.load_gather(tab_v, [ja])
                        vb = plsc.load_gather(tab_v, [jb])
                        out_v[dd, b, pl.ds(0, L)] = va
                        out_v[dd, b, pl.ds(F - L, L)] = vb
                        if dd + 1 < DBLK:
                            ja = ja + F * V
                            jb = jb + F * V

                copies = [
                    pltpu.async_copy(
                        out_v.at[dd],
                        out_hbm.at[pl.ds((dlo + dd) * B + b0, CHUNK), :],
                        sem_out,
                    )
                    for dd in range(DBLK)
                ]
                for cp in copies:
                    cp.wait()
            return 0

        lax.fori_loop(0, chunks_per_w, chunk_body, 0)

    return k(jbase, tables_flat)


def kernel(inputs, tables):
    jbase = (inputs.astype(jnp.int32)
             + (jnp.arange(F, dtype=jnp.int32) * V)[None, :])
    jbase = jbase.reshape(B * F)
    tables_flat = jnp.transpose(tables, (2, 0, 1)).reshape(TABLE_W)
    out2 = _sc_embed(jbase, tables_flat, 32)
    return out2.reshape(D, B, F)
